# baseline (speedup 1.0000x reference)
import jax
import jax.numpy as jnp
from jax.experimental import pallas as pl

_DIM = 1000
_R = 512


def _onehot_body(idx_ref, out_ref):
    idx = idx_ref[...]
    iota = jax.lax.broadcasted_iota(jnp.int32, (1, _DIM, _R), 1)
    out_ref[...] = (iota == idx).astype(jnp.float32)


def kernel(tensor):
    n0, n1 = tensor.shape
    idx_t = tensor.astype(jnp.int32).T.reshape(n1, 1, n0)
    out_phys = pl.pallas_call(
        _onehot_body,
        grid=(n1, n0 // _R),
        in_specs=[pl.BlockSpec((1, 1, _R), lambda c, r: (c, 0, r))],
        out_specs=pl.BlockSpec((1, _DIM, _R), lambda c, r: (c, 0, r)),
        out_shape=jax.ShapeDtypeStruct((n1, _DIM, n0), jnp.float32),
    )(idx_t)
    return jnp.transpose(out_phys, (2, 0, 1))


# speedup vs baseline: 1.3044x; 1.3044x over previous
import jax
import jax.numpy as jnp
from jax.experimental import pallas as pl

_DIM = 1000
_D = 200


def _onehot_body(idx_ref, out_ref):
    j = pl.program_id(1)
    n0 = idx_ref.shape[2]
    idx = idx_ref[...]
    iota = jax.lax.broadcasted_iota(jnp.int32, (1, _D, n0), 1) + j * _D
    out_ref[...] = (iota == idx).astype(jnp.float32)


def kernel(tensor):
    n0, n1 = tensor.shape
    idx_t = tensor.astype(jnp.int32).T.reshape(n1, 1, n0)
    out_phys = pl.pallas_call(
        _onehot_body,
        grid=(n1, _DIM // _D),
        in_specs=[pl.BlockSpec((1, 1, n0), lambda c, j: (c, 0, 0))],
        out_specs=pl.BlockSpec((1, _D, n0), lambda c, j: (c, j, 0)),
        out_shape=jax.ShapeDtypeStruct((n1, _DIM, n0), jnp.float32),
    )(idx_t)
    return jnp.transpose(out_phys, (2, 0, 1))
